# baseline (speedup 1.0000x reference)
import jax
import jax.numpy as jnp
from jax import lax
from jax.experimental import pallas as pl
from jax.experimental.pallas import tpu as pltpu
from jax.experimental.pallas import tpu_sc as plsc

N = 100000
D = 128
S = 256
NC = 2
NS = 16
NW = NC * NS
RPW = N // NW
CHUNK = 125
NCHUNK = RPW // CHUNK
BB = 3152
NPAD = 100096
ACC = S * D
DEN = S * 16


def _sc_body(x_hbm, batch_hbm, w_hbm, b_hbm, pacc_hbm, pden_hbm,
             xbuf, wv, bv, bb, acc, den):
    cid = lax.axis_index("c")
    sid = lax.axis_index("s")
    wid = sid * NC + cid
    row0 = wid * RPW
    al = (row0 // 8) * 8
    extra = row0 - al

    pltpu.sync_copy(batch_hbm.at[pl.ds(al, BB)], bb)
    pltpu.sync_copy(w_hbm, wv)
    pltpu.sync_copy(b_hbm, bv)

    zero16 = jnp.zeros((16,), jnp.float32)

    def zacc(i, carry):
        acc[pl.ds(i * 16, 16)] = zero16
        return carry

    lax.fori_loop(0, ACC // 16, zacc, 0)

    def zden(i, carry):
        den[pl.ds(i * 16, 16)] = zero16
        return carry

    lax.fori_loop(0, DEN // 16, zden, 0)

    wvecs = [wv[pl.ds(16 * j, 16)] for j in range(D // 16)]
    bvec = bv[...]
    lane = lax.iota(jnp.int32, 16)

    def chunk_body(c, carry):
        pltpu.sync_copy(x_hbm.at[pl.ds((row0 + c * CHUNK) * D, CHUNK * D)], xbuf)

        def row_body(r, carry2):
            seg = bb[pl.ds(extra + c * CHUNK + r, 16)][0]
            xv = [xbuf[pl.ds(r * D + 16 * j, 16)] for j in range(D // 16)]
            part = xv[0] * wvecs[0]
            for j in range(1, D // 16):
                part = part + xv[j] * wvecs[j]
            s = part
            for k in (8, 4, 2, 1):
                s = s + jnp.take(s, lane ^ k, axis=0)
            ev = jnp.exp(bvec + s)
            base = seg * D
            for j in range(D // 16):
                plsc.addupdate(acc.at[pl.ds(base + 16 * j, 16)], ev * xv[j])
            plsc.addupdate(den.at[pl.ds(seg * 16, 16)], ev)
            return carry2

        lax.fori_loop(0, CHUNK, row_body, 0)
        return carry

    lax.fori_loop(0, NCHUNK, chunk_body, 0)

    pltpu.sync_copy(acc, pacc_hbm.at[pl.ds(wid * ACC, ACC)])
    pltpu.sync_copy(den, pden_hbm.at[pl.ds(wid * DEN, DEN)])


def _sc_pool(xflat, batch_pad, wflat, b16):
    mesh = plsc.VectorSubcoreMesh(core_axis_name="c", subcore_axis_name="s")
    f = pl.kernel(
        _sc_body,
        out_type=(
            jax.ShapeDtypeStruct((NW * ACC,), jnp.float32),
            jax.ShapeDtypeStruct((NW * DEN,), jnp.float32),
        ),
        mesh=mesh,
        scratch_types=[
            pltpu.VMEM((CHUNK * D,), jnp.float32),
            pltpu.VMEM((D,), jnp.float32),
            pltpu.VMEM((16,), jnp.float32),
            pltpu.VMEM((BB,), jnp.int32),
            pltpu.VMEM((ACC,), jnp.float32),
            pltpu.VMEM((DEN,), jnp.float32),
        ],
    )
    return f(xflat, batch_pad, wflat, b16)


def _ep_body(pacc_ref, pden_ref, out_ref):
    s = jnp.sum(pacc_ref[...], axis=0)
    dn = jnp.sum(pden_ref[...], axis=0)
    d0 = dn[:, 0:1]
    out_ref[...] = s / jnp.maximum(d0, 1e-30)


def kernel(x, batch, W, b):
    batch_pad = jnp.pad(batch.astype(jnp.int32), (0, NPAD - N))
    wflat = W.reshape(D).astype(jnp.float32)
    b16 = jnp.broadcast_to(b.astype(jnp.float32), (16,))
    pacc, pden = _sc_pool(x.reshape(N * D), batch_pad, wflat, b16)
    out = pl.pallas_call(
        _ep_body,
        out_shape=jax.ShapeDtypeStruct((S, D), jnp.float32),
    )(pacc.reshape(NW, S, D), pden.reshape(NW, S, 16))
    return out


# speedup vs baseline: 1.2227x; 1.2227x over previous
import jax
import jax.numpy as jnp
from jax import lax
from jax.experimental import pallas as pl
from jax.experimental.pallas import tpu as pltpu
from jax.experimental.pallas import tpu_sc as plsc

N = 100000
D = 128
S = 256
NC = 2
NS = 16
NW = NC * NS
RPW = N // NW
CHUNK = 125
NCHUNK = RPW // CHUNK
UNROLL = 5
BB = 3152
NPAD = 100096
ACC = S * D
DEN = S * 16


def _sc_body(x_hbm, batch_hbm, w_hbm, b_hbm, pacc_hbm, pden_hbm,
             xb0, xb1, wv, bv, bb, acc, den, sem0, sem1):
    cid = lax.axis_index("c")
    sid = lax.axis_index("s")
    wid = sid * NC + cid
    row0 = wid * RPW
    al = (row0 // 8) * 8
    extra = row0 - al

    pltpu.sync_copy(batch_hbm.at[pl.ds(al, BB)], bb)
    pltpu.sync_copy(w_hbm, wv)
    pltpu.sync_copy(b_hbm, bv)

    zero16 = jnp.zeros((16,), jnp.float32)

    def zacc(i, carry):
        acc[pl.ds(i * 16, 16)] = zero16
        return carry

    lax.fori_loop(0, ACC // 16, zacc, 0)

    def zden(i, carry):
        den[pl.ds(i * 16, 16)] = zero16
        return carry

    lax.fori_loop(0, DEN // 16, zden, 0)

    wvecs = [wv[pl.ds(16 * j, 16)] for j in range(D // 16)]
    bvec = bv[...]
    lane = lax.iota(jnp.int32, 16)

    def _slice(c):
        return x_hbm.at[pl.ds((row0 + c * CHUNK) * D, CHUNK * D)]

    def start(c, buf, sem):
        pltpu.async_copy(_slice(c), buf, sem)

    def wait(c, buf, sem):
        pltpu.make_async_copy(_slice(c), buf, sem).wait()

    def row(buf, c, r):
        seg = bb[pl.ds(extra + c * CHUNK + r, 16)][0]
        xv = [buf[pl.ds(r * D + 16 * j, 16)] for j in range(D // 16)]
        prods = [xv[j] * wvecs[j] for j in range(D // 16)]
        while len(prods) > 1:
            prods = [prods[i] + prods[i + 1] for i in range(0, len(prods), 2)]
        s = prods[0]
        for k in (8, 4, 2, 1):
            s = s + jnp.take(s, lane ^ k, axis=0)
        ev = jnp.exp(bvec + s)
        base = seg * D
        for j in range(D // 16):
            plsc.addupdate(acc.at[pl.ds(base + 16 * j, 16)], ev * xv[j])
        plsc.addupdate(den.at[pl.ds(seg * 16, 16)], ev)

    def process(buf, c):
        def rb(q, carry):
            for u in range(UNROLL):
                row(buf, c, q * UNROLL + u)
            return carry

        lax.fori_loop(0, CHUNK // UNROLL, rb, 0)

    start(0, xb0, sem0)

    def pair_body(i, carry):
        c0 = 2 * i
        start(c0 + 1, xb1, sem1)
        wait(c0, xb0, sem0)
        process(xb0, c0)
        start(c0 + 2, xb0, sem0)
        wait(c0 + 1, xb1, sem1)
        process(xb1, c0 + 1)
        return carry

    lax.fori_loop(0, (NCHUNK - 1) // 2, pair_body, 0)
    wait(NCHUNK - 1, xb0, sem0)
    process(xb0, NCHUNK - 1)

    pltpu.sync_copy(acc, pacc_hbm.at[pl.ds(wid * ACC, ACC)])
    pltpu.sync_copy(den, pden_hbm.at[pl.ds(wid * DEN, DEN)])


def _sc_pool(xflat, batch_pad, wflat, b16):
    mesh = plsc.VectorSubcoreMesh(core_axis_name="c", subcore_axis_name="s")
    f = pl.kernel(
        _sc_body,
        out_type=(
            jax.ShapeDtypeStruct((NW * ACC,), jnp.float32),
            jax.ShapeDtypeStruct((NW * DEN,), jnp.float32),
        ),
        mesh=mesh,
        scratch_types=[
            pltpu.VMEM((CHUNK * D,), jnp.float32),
            pltpu.VMEM((CHUNK * D,), jnp.float32),
            pltpu.VMEM((D,), jnp.float32),
            pltpu.VMEM((16,), jnp.float32),
            pltpu.VMEM((BB,), jnp.int32),
            pltpu.VMEM((ACC,), jnp.float32),
            pltpu.VMEM((DEN,), jnp.float32),
            pltpu.SemaphoreType.DMA,
            pltpu.SemaphoreType.DMA,
        ],
    )
    return f(xflat, batch_pad, wflat, b16)


def _ep_body(pacc_ref, pden_ref, out_ref):
    s = jnp.sum(pacc_ref[...], axis=0)
    dn = jnp.sum(pden_ref[...], axis=0)
    d0 = dn[:, 0:1]
    out_ref[...] = s / jnp.maximum(d0, 1e-30)


def kernel(x, batch, W, b):
    batch_pad = jnp.pad(batch.astype(jnp.int32), (0, NPAD - N))
    wflat = W.reshape(D).astype(jnp.float32)
    b16 = jnp.broadcast_to(b.astype(jnp.float32), (16,))
    pacc, pden = _sc_pool(x.reshape(N * D), batch_pad, wflat, b16)
    out = pl.pallas_call(
        _ep_body,
        out_shape=jax.ShapeDtypeStruct((S, D), jnp.float32),
    )(pacc.reshape(NW, S, D), pden.reshape(NW, S, 16))
    return out


# speedup vs baseline: 2.6773x; 2.1896x over previous
import jax
import jax.numpy as jnp
from jax import lax
from jax.experimental import pallas as pl
from jax.experimental.pallas import tpu as pltpu
from jax.experimental.pallas import tpu_sc as plsc

N = 100000
D = 128
S = 256

NC = 2
NS = 16
NW = NC * NS
CHUNK = 125
NCHUNK = 7
RPW = CHUNK * NCHUNK
K = NW * RPW
M = N - K
UNROLL = 5
BB = 912
NPAD = 100096
ACC = S * D
DEN = S * 16

B = 2000
MB = M // B


def _sc_body(x_hbm, batch_hbm, w_hbm, b_hbm, pacc_hbm, pden_hbm,
             xb0, xb1, wv, bv, bb, acc, den, sem0, sem1):
    cid = lax.axis_index("c")
    sid = lax.axis_index("s")
    wid = sid * NC + cid
    row0 = M + wid * RPW
    al = (row0 // 8) * 8
    extra = row0 - al

    pltpu.sync_copy(batch_hbm.at[pl.ds(al, BB)], bb)
    pltpu.sync_copy(w_hbm, wv)
    pltpu.sync_copy(b_hbm, bv)

    zero16 = jnp.zeros((16,), jnp.float32)

    def zacc(i, carry):
        acc[pl.ds(i * 16, 16)] = zero16
        return carry

    lax.fori_loop(0, ACC // 16, zacc, 0)

    def zden(i, carry):
        den[pl.ds(i * 16, 16)] = zero16
        return carry

    lax.fori_loop(0, DEN // 16, zden, 0)

    wvecs = [wv[pl.ds(16 * j, 16)] for j in range(D // 16)]
    bvec = bv[...]
    lane = lax.iota(jnp.int32, 16)

    def _slice(c):
        return x_hbm.at[pl.ds((row0 + c * CHUNK) * D, CHUNK * D)]

    def start(c, buf, sem):
        pltpu.async_copy(_slice(c), buf, sem)

    def wait(c, buf, sem):
        pltpu.make_async_copy(_slice(c), buf, sem).wait()

    def group(buf, c, q):
        r0 = q * UNROLL
        segs = [bb[pl.ds(extra + c * CHUNK + r0 + u, 16)][0] for u in range(UNROLL)]
        xvs = [[buf[pl.ds((r0 + u) * D + 16 * j, 16)] for j in range(D // 16)]
               for u in range(UNROLL)]
        parts = []
        for u in range(UNROLL):
            prods = [xvs[u][j] * wvecs[j] for j in range(D // 16)]
            while len(prods) > 1:
                prods = [prods[i] + prods[i + 1] for i in range(0, len(prods), 2)]
            parts.append(prods[0])
        for k in (8, 4, 2, 1):
            parts = [p + jnp.take(p, lane ^ k, axis=0) for p in parts]
        evs = [jnp.exp(bvec + p) for p in parts]
        for u in range(UNROLL):
            base = segs[u] * D
            for j in range(D // 16):
                plsc.addupdate(acc.at[pl.ds(base + 16 * j, 16)], evs[u] * xvs[u][j])
            plsc.addupdate(den.at[pl.ds(segs[u] * 16, 16)], evs[u])

    def process(buf, c):
        def rb(q, carry):
            group(buf, c, q)
            return carry

        lax.fori_loop(0, CHUNK // UNROLL, rb, 0)

    start(0, xb0, sem0)

    def pair_body(i, carry):
        c0 = 2 * i
        start(c0 + 1, xb1, sem1)
        wait(c0, xb0, sem0)
        process(xb0, c0)
        start(c0 + 2, xb0, sem0)
        wait(c0 + 1, xb1, sem1)
        process(xb1, c0 + 1)
        return carry

    lax.fori_loop(0, (NCHUNK - 1) // 2, pair_body, 0)
    wait(NCHUNK - 1, xb0, sem0)
    process(xb0, NCHUNK - 1)

    pltpu.sync_copy(acc, pacc_hbm.at[pl.ds(wid * ACC, ACC)])
    pltpu.sync_copy(den, pden_hbm.at[pl.ds(wid * DEN, DEN)])


def _sc_pool(xflat, batch_pad, wflat, b16):
    mesh = plsc.VectorSubcoreMesh(core_axis_name="c", subcore_axis_name="s")
    f = pl.kernel(
        _sc_body,
        out_type=(
            jax.ShapeDtypeStruct((NW * ACC,), jnp.float32),
            jax.ShapeDtypeStruct((NW * DEN,), jnp.float32),
        ),
        mesh=mesh,
        scratch_types=[
            pltpu.VMEM((CHUNK * D,), jnp.float32),
            pltpu.VMEM((CHUNK * D,), jnp.float32),
            pltpu.VMEM((D,), jnp.float32),
            pltpu.VMEM((16,), jnp.float32),
            pltpu.VMEM((BB,), jnp.int32),
            pltpu.VMEM((ACC,), jnp.float32),
            pltpu.VMEM((DEN,), jnp.float32),
            pltpu.SemaphoreType.DMA,
            pltpu.SemaphoreType.DMA,
        ],
    )
    return f(xflat, batch_pad, wflat, b16)


def _tc_body(batch_ref, x_ref, w_ref, b_ref, num_out, den_out, num_ref, den_ref):
    i = pl.program_id(0)

    @pl.when(i == 0)
    def _():
        num_ref[...] = jnp.zeros_like(num_ref)
        den_ref[...] = jnp.zeros_like(den_ref)

    x = x_ref[...]
    w = w_ref[...]
    g = jnp.sum(x * w, axis=1, keepdims=True) + b_ref[0, 0]
    e = jnp.exp(g)
    bv = batch_ref[0]
    ids = jax.lax.broadcasted_iota(jnp.int32, (S, B), 0)
    ohb = ids == bv
    oh = ohb.astype(jnp.bfloat16)
    xe = (x * e).astype(jnp.bfloat16)
    num_ref[...] += jax.lax.dot(oh, xe, preferred_element_type=jnp.float32)
    erow = jnp.broadcast_to(e.reshape(1, B), (S, B))
    den_ref[...] += jnp.sum(jnp.where(ohb, erow, 0.0), axis=1, keepdims=True)

    @pl.when(i == MB - 1)
    def _():
        num_out[...] = num_ref[...]
        den_out[...] = den_ref[...]


def _tc_pool(x, batch3, W, b2):
    return pl.pallas_call(
        _tc_body,
        grid=(MB,),
        in_specs=[
            pl.BlockSpec((1, 1, B), lambda i: (i, 0, 0)),
            pl.BlockSpec((B, D), lambda i: (i, 0)),
            pl.BlockSpec((1, D), lambda i: (0, 0)),
            pl.BlockSpec((1, 1), lambda i: (0, 0)),
        ],
        out_specs=[
            pl.BlockSpec((S, D), lambda i: (0, 0)),
            pl.BlockSpec((S, 1), lambda i: (0, 0)),
        ],
        out_shape=[
            jax.ShapeDtypeStruct((S, D), jnp.float32),
            jax.ShapeDtypeStruct((S, 1), jnp.float32),
        ],
        scratch_shapes=[
            pltpu.VMEM((S, D), jnp.float32),
            pltpu.VMEM((S, 1), jnp.float32),
        ],
        compiler_params=pltpu.CompilerParams(
            dimension_semantics=("arbitrary",),
        ),
    )(batch3, x, W, b2)


def _ep_body(pacc_ref, pden_ref, num_ref, den_ref, out_ref):
    s = num_ref[...] + jnp.sum(pacc_ref[...], axis=0)
    dn = den_ref[...] + jnp.sum(pden_ref[..., 0:1], axis=0)
    out_ref[...] = s / jnp.maximum(dn, 1e-30)


def kernel(x, batch, W, b):
    batch_pad = jnp.pad(batch.astype(jnp.int32), (0, NPAD - N))
    wflat = W.reshape(D).astype(jnp.float32)
    b16 = jnp.broadcast_to(b.astype(jnp.float32), (16,))
    batch3 = batch.astype(jnp.int32).reshape(N // B, 1, B)
    b2 = b.reshape(1, 1).astype(jnp.float32)
    pacc, pden = _sc_pool(x.reshape(N * D), batch_pad, wflat, b16)
    num_tc, den_tc = _tc_pool(x, batch3, W, b2)
    out = pl.pallas_call(
        _ep_body,
        out_shape=jax.ShapeDtypeStruct((S, D), jnp.float32),
    )(pacc.reshape(NW, S, D), pden.reshape(NW, S, 16), num_tc, den_tc)
    return out
